# baseline (speedup 1.0000x reference)
import dataclasses
import functools

import jax
import jax.numpy as jnp
from jax.experimental import pallas as pl
from jax.experimental.pallas import tpu as pltpu
from jax.experimental.pallas import tpu_sc as plsc

N = 8192
K = 16
RB = 256
IMAX = 2**31 - 1


def _oems_pairs(n):
    pairs = []

    def merge(lo, hi, r):
        step = r * 2
        if step < hi - lo:
            merge(lo, hi, step)
            merge(lo + r, hi, step)
            pairs.extend((i, i + r) for i in range(lo + r, hi - r, step))
        else:
            pairs.append((lo, lo + r))

    def sort(lo, hi):
        if hi - lo >= 1:
            mid = lo + (hi - lo) // 2
            sort(lo, mid)
            sort(mid + 1, hi)
            merge(lo, hi, 1)

    sort(0, n - 1)
    return pairs


_SORT16 = _oems_pairs(16)


def _sortnet16(vals):
    vals = list(vals)
    for i, j in _SORT16:
        a, b = vals[i], vals[j]
        vals[i] = jnp.minimum(a, b)
        vals[j] = jnp.maximum(a, b)
    return vals


def _merge_low16(a, b):
    lo = [jnp.minimum(a[i], b[15 - i]) for i in range(16)]
    d = 8
    while d >= 1:
        for base in range(0, 16, 2 * d):
            for i in range(base, base + d):
                x, y = lo[i], lo[i + d]
                lo[i] = jnp.minimum(x, y)
                lo[i + d] = jnp.maximum(x, y)
        d //= 2
    return lo


def _knn_body(cols_ref, cand_ref, out_ref):
    xi = cols_ref[0:1, :]
    yi = cols_ref[1:2, :]
    zi = cols_ref[2:3, :]
    c_iota = jax.lax.broadcasted_iota(jnp.int32, (8, 128), 0)

    def batch_fn(b, state):
        slab = cand_ref[pl.ds(b * 128, 128), :]
        batch = []
        for t in range(16):
            xj = slab[8 * t:8 * t + 8, 0:1]
            yj = slab[8 * t:8 * t + 8, 1:2]
            zj = slab[8 * t:8 * t + 8, 2:3]
            dx = xj - xi
            dy = yj - yi
            dz = zj - zi
            d2 = dx * dx + dy * dy + dz * dz
            batch.append(d2 * N + (b * 128 + 8 * t) + c_iota)
        batch = _sortnet16(batch)
        return tuple(_merge_low16(list(state), batch))

    init = tuple(jnp.full((8, 128), IMAX, jnp.int32) for _ in range(16))
    state = list(jax.lax.fori_loop(0, N // 128, batch_fn, init))
    w = 8
    while w > 1:
        h = w // 2
        state = _merge_low16([p[0:h, :] for p in state],
                             [p[h:w, :] for p in state])
        w = h
    outT = jnp.concatenate(state, axis=0)
    out_ref[...] = outT & (N - 1)


def _sc_gather(pc, idx_flat, interpret=False):
    info = plsc.get_sparse_core_info()
    nw = info.num_cores * info.num_subcores
    b_per_w = idx_flat.shape[0] // nw
    mesh = plsc.VectorSubcoreMesh(core_axis_name="c", subcore_axis_name="s")
    cp = pltpu.CompilerParams()
    if "needs_layout_passes" in pltpu.CompilerParams.__dataclass_fields__:
        cp = dataclasses.replace(cp, needs_layout_passes=False)

    @functools.partial(
        pl.kernel, mesh=mesh,
        out_type=jax.ShapeDtypeStruct((idx_flat.shape[0],), jnp.int32),
        scratch_types=[
            pltpu.VMEM((N,), jnp.int32),
            pltpu.VMEM((b_per_w,), jnp.int32),
            pltpu.VMEM((b_per_w,), jnp.int32),
        ],
        compiler_params=cp,
        interpret=interpret,
    )
    def k(pc_hbm, idx_hbm, out_hbm, pc_v, idx_v, out_v):
        wid = jax.lax.axis_index("s") * info.num_cores + jax.lax.axis_index("c")
        base = wid * b_per_w
        pltpu.sync_copy(pc_hbm, pc_v)
        pltpu.sync_copy(idx_hbm.at[pl.ds(base, b_per_w)], idx_v)

        @pl.loop(0, b_per_w, step=16)
        def _(i):
            j16 = idx_v[pl.ds(i, 16)]
            out_v[pl.ds(i, 16)] = plsc.load_gather(pc_v, [j16])

        pltpu.sync_copy(out_v, out_hbm.at[pl.ds(base, b_per_w)])

    return k(pc, idx_flat)


MB = 1024


def _mlp1_body(pc_ref, w1_ref, b1_ref, h_ref, sums_ref):
    pc = pc_ref[...]
    x = pc >> 14
    y = (pc >> 7) & 127
    z = pc & 127
    relx = (x[:, 1:] - x[:, 0:1]).astype(jnp.float32)
    rely = (y[:, 1:] - y[:, 0:1]).astype(jnp.float32)
    relz = (z[:, 1:] - z[:, 0:1]).astype(jnp.float32)
    rel = jnp.concatenate([relx, rely, relz], axis=1)
    h = jax.lax.dot_general(
        rel, w1_ref[...], (((1,), (0,)), ((), ())),
        precision=jax.lax.Precision.HIGHEST,
        preferred_element_type=jnp.float32) + b1_ref[...]
    h_ref[...] = h
    ps = jnp.sum(h, axis=0, keepdims=True)
    ps2 = jnp.sum(h * h, axis=0, keepdims=True)
    part = jnp.concatenate([ps, ps2], axis=0)

    @pl.when(pl.program_id(0) == 0)
    def _():
        sums_ref[...] = part

    @pl.when(pl.program_id(0) != 0)
    def _():
        sums_ref[...] += part


def _mlp2_body(sums_ref, h_ref, gamma_ref, beta_ref, w2_ref, b2_ref, out_ref):
    inv_n = 1.0 / N
    mean = sums_ref[0:1, :] * inv_n
    var = sums_ref[1:2, :] * inv_n - mean * mean
    scale = gamma_ref[...] / jnp.sqrt(var + 1e-5)
    shift = beta_ref[...] - mean * scale
    hn = jnp.maximum(h_ref[...] * scale + shift, 0.0)
    out_ref[...] = jax.lax.dot_general(
        hn, w2_ref[...], (((1,), (0,)), ((), ())),
        precision=jax.lax.Precision.HIGHEST,
        preferred_element_type=jnp.float32) + b2_ref[...]


def _run(coords, colsT, pc, W1p, b1, gamma, beta, W2, b2, interpret=False):
    npf = W1p.shape[1]
    idxT = pl.pallas_call(
        _knn_body,
        grid=(N // 128,),
        in_specs=[
            pl.BlockSpec((3, 128), lambda i: (0, i)),
            pl.BlockSpec((N, 3), lambda i: (0, 0)),
        ],
        out_specs=pl.BlockSpec((K, 128), lambda i: (0, i)),
        out_shape=jax.ShapeDtypeStruct((K, N), jnp.int32),
        compiler_params=pltpu.CompilerParams(
            dimension_semantics=("parallel",)),
        interpret=interpret,
    )(colsT, coords)

    idx = idxT.T
    selpc = _sc_gather(pc.reshape(N), idx.reshape(N * K),
                       interpret=interpret).reshape(N, K)

    nblk = N // MB
    h, sums = pl.pallas_call(
        _mlp1_body,
        grid=(nblk,),
        in_specs=[
            pl.BlockSpec((MB, K), lambda i: (i, 0)),
            pl.BlockSpec((3 * (K - 1), npf), lambda i: (0, 0)),
            pl.BlockSpec((1, npf), lambda i: (0, 0)),
        ],
        out_specs=[
            pl.BlockSpec((MB, npf), lambda i: (i, 0)),
            pl.BlockSpec((2, npf), lambda i: (0, 0)),
        ],
        out_shape=[
            jax.ShapeDtypeStruct((N, npf), jnp.float32),
            jax.ShapeDtypeStruct((2, npf), jnp.float32),
        ],
        interpret=interpret,
    )(selpc, W1p, b1)

    out = pl.pallas_call(
        _mlp2_body,
        grid=(nblk,),
        in_specs=[
            pl.BlockSpec((2, npf), lambda i: (0, 0)),
            pl.BlockSpec((MB, npf), lambda i: (i, 0)),
            pl.BlockSpec((1, npf), lambda i: (0, 0)),
            pl.BlockSpec((1, npf), lambda i: (0, 0)),
            pl.BlockSpec((npf, npf), lambda i: (0, 0)),
            pl.BlockSpec((1, npf), lambda i: (0, 0)),
        ],
        out_specs=pl.BlockSpec((MB, npf), lambda i: (i, 0)),
        out_shape=jax.ShapeDtypeStruct((N, npf), jnp.float32),
        interpret=interpret,
    )(sums, h, gamma, beta, W2, b2)
    return out


def kernel(indices, W1, b1, gamma, beta, W2, b2):
    coords = indices[:, 1:].astype(jnp.int32)
    colsT = coords.T
    pc = ((coords[:, 0] * 128 + coords[:, 1]) * 128
          + coords[:, 2]).reshape(1, N)
    npf = W1.shape[1]
    W1p = W1.reshape(K - 1, 3, npf).transpose(1, 0, 2).reshape(3 * (K - 1), npf)
    return _run(coords, colsT, pc, W1p,
                b1.reshape(1, npf), gamma.reshape(1, npf),
                beta.reshape(1, npf), W2, b2.reshape(1, npf))


# speedup vs baseline: 2.8855x; 2.8855x over previous
import dataclasses
import functools

import jax
import jax.numpy as jnp
from jax.experimental import pallas as pl
from jax.experimental.pallas import tpu as pltpu
from jax.experimental.pallas import tpu_sc as plsc

N = 8192
K = 16


def _oems_pairs(n):
    pairs = []

    def merge(lo, hi, r):
        step = r * 2
        if step < hi - lo:
            merge(lo, hi, step)
            merge(lo + r, hi, step)
            pairs.extend((i, i + r) for i in range(lo + r, hi - r, step))
        else:
            pairs.append((lo, lo + r))

    def sort(lo, hi):
        if hi - lo >= 1:
            mid = lo + (hi - lo) // 2
            sort(lo, mid)
            sort(mid + 1, hi)
            merge(lo, hi, 1)

    sort(0, n - 1)
    return pairs


_SORT16 = _oems_pairs(16)


def _sortnet16(vals):
    vals = list(vals)
    for i, j in _SORT16:
        a, b = vals[i], vals[j]
        vals[i] = jnp.minimum(a, b)
        vals[j] = jnp.maximum(a, b)
    return vals


def _merge_low16(a, b):
    lo = [jnp.minimum(a[i], b[15 - i]) for i in range(16)]
    d = 8
    while d >= 1:
        for base in range(0, 16, 2 * d):
            for i in range(base, base + d):
                x, y = lo[i], lo[i + d]
                lo[i] = jnp.minimum(x, y)
                lo[i + d] = jnp.maximum(x, y)
        d //= 2
    return lo


def _knn_body(cols_ref, xe_ref, ye_ref, ze_ref, out_ref):
    xi = cols_ref[0:1, :]
    yi = cols_ref[1:2, :]
    zi = cols_ref[2:3, :]
    c_iota = jax.lax.broadcasted_iota(jnp.int32, (8, 128), 0)

    def make_batch(b):
        batch = []
        for t in range(16):
            sl = pl.ds(b * 128 + 8 * t, 8)
            dx = xe_ref[sl, :] - xi
            dy = ye_ref[sl, :] - yi
            dz = ze_ref[sl, :] - zi
            d2 = dx * dx + dy * dy + dz * dz
            ki = d2 * N + (2**23 + b * 128 + 8 * t) + c_iota
            batch.append(jax.lax.bitcast_convert_type(ki, jnp.float32))
        return _sortnet16(batch)

    def batch_fn(i, state):
        b01 = _merge_low16(make_batch(8 * i), make_batch(8 * i + 1))
        b23 = _merge_low16(make_batch(8 * i + 2), make_batch(8 * i + 3))
        b45 = _merge_low16(make_batch(8 * i + 4), make_batch(8 * i + 5))
        b67 = _merge_low16(make_batch(8 * i + 6), make_batch(8 * i + 7))
        b = _merge_low16(_merge_low16(b01, b23), _merge_low16(b45, b67))
        return tuple(_merge_low16(list(state), b))

    init = tuple(jnp.full((8, 128), jnp.finfo(jnp.float32).max, jnp.float32)
                 for _ in range(16))
    state = list(jax.lax.fori_loop(0, N // 1024, batch_fn, init))
    w = 8
    while w > 1:
        h = w // 2
        state = _merge_low16([p[0:h, :] for p in state],
                             [p[h:w, :] for p in state])
        w = h
    outT = jax.lax.bitcast_convert_type(
        jnp.concatenate(state, axis=0), jnp.int32)
    out_ref[...] = outT & (N - 1)


def _sc_gather(pc, idx_flat, interpret=False):
    info = plsc.get_sparse_core_info()
    nw = info.num_cores * info.num_subcores
    b_per_w = idx_flat.shape[0] // nw
    mesh = plsc.VectorSubcoreMesh(core_axis_name="c", subcore_axis_name="s")
    cp = pltpu.CompilerParams()
    if "needs_layout_passes" in pltpu.CompilerParams.__dataclass_fields__:
        cp = dataclasses.replace(cp, needs_layout_passes=False)

    @functools.partial(
        pl.kernel, mesh=mesh,
        out_type=jax.ShapeDtypeStruct((idx_flat.shape[0],), jnp.int32),
        scratch_types=[
            pltpu.VMEM((N,), jnp.int32),
            pltpu.VMEM((b_per_w,), jnp.int32),
            pltpu.VMEM((b_per_w,), jnp.int32),
        ],
        compiler_params=cp,
        interpret=interpret,
    )
    def k(pc_hbm, idx_hbm, out_hbm, pc_v, idx_v, out_v):
        wid = jax.lax.axis_index("s") * info.num_cores + jax.lax.axis_index("c")
        base = wid * b_per_w
        pltpu.sync_copy(pc_hbm, pc_v)
        pltpu.sync_copy(idx_hbm.at[pl.ds(base, b_per_w)], idx_v)

        @pl.loop(0, b_per_w, step=16)
        def _(i):
            j16 = idx_v[pl.ds(i, 16)]
            out_v[pl.ds(i, 16)] = plsc.load_gather(pc_v, [j16])

        pltpu.sync_copy(out_v, out_hbm.at[pl.ds(base, b_per_w)])

    return k(pc, idx_flat)


MB = 1024


def _mlp1_body(pc_ref, w1_ref, b1_ref, h_ref, sums_ref):
    pc = pc_ref[...]
    x = pc >> 14
    y = (pc >> 7) & 127
    z = pc & 127
    relx = (x[:, 1:] - x[:, 0:1]).astype(jnp.float32)
    rely = (y[:, 1:] - y[:, 0:1]).astype(jnp.float32)
    relz = (z[:, 1:] - z[:, 0:1]).astype(jnp.float32)
    rel = jnp.concatenate([relx, rely, relz], axis=1)
    h = jax.lax.dot_general(
        rel, w1_ref[...], (((1,), (0,)), ((), ())),
        preferred_element_type=jnp.float32) + b1_ref[...]
    h_ref[...] = h
    ps = jnp.sum(h, axis=0, keepdims=True)
    ps2 = jnp.sum(h * h, axis=0, keepdims=True)
    part = jnp.concatenate([ps, ps2], axis=0)

    @pl.when(pl.program_id(0) == 0)
    def _():
        sums_ref[...] = part

    @pl.when(pl.program_id(0) != 0)
    def _():
        sums_ref[...] += part


def _mlp2_body(sums_ref, h_ref, gamma_ref, beta_ref, w2_ref, b2_ref, out_ref):
    inv_n = 1.0 / N
    mean = sums_ref[0:1, :] * inv_n
    var = sums_ref[1:2, :] * inv_n - mean * mean
    scale = gamma_ref[...] / jnp.sqrt(var + 1e-5)
    shift = beta_ref[...] - mean * scale
    hn = jnp.maximum(h_ref[...] * scale + shift, 0.0)
    out_ref[...] = jax.lax.dot_general(
        hn, w2_ref[...], (((1,), (0,)), ((), ())),
        preferred_element_type=jnp.float32) + b2_ref[...]


def _run(coords, colsT, pc, W1p, b1, gamma, beta, W2, b2, interpret=False):
    npf = W1p.shape[1]
    xe = jnp.broadcast_to(coords[:, 0:1], (N, 128))
    ye = jnp.broadcast_to(coords[:, 1:2], (N, 128))
    ze = jnp.broadcast_to(coords[:, 2:3], (N, 128))
    idxT = pl.pallas_call(
        _knn_body,
        grid=(N // 128,),
        in_specs=[
            pl.BlockSpec((3, 128), lambda i: (0, i)),
            pl.BlockSpec((N, 128), lambda i: (0, 0)),
            pl.BlockSpec((N, 128), lambda i: (0, 0)),
            pl.BlockSpec((N, 128), lambda i: (0, 0)),
        ],
        out_specs=pl.BlockSpec((K, 128), lambda i: (0, i)),
        out_shape=jax.ShapeDtypeStruct((K, N), jnp.int32),
        compiler_params=pltpu.CompilerParams(
            dimension_semantics=("parallel",)),
        interpret=interpret,
    )(colsT, xe, ye, ze)

    idx = idxT.T
    selpc = _sc_gather(pc.reshape(N), idx.reshape(N * K),
                       interpret=interpret).reshape(N, K)

    nblk = N // MB
    h, sums = pl.pallas_call(
        _mlp1_body,
        grid=(nblk,),
        in_specs=[
            pl.BlockSpec((MB, K), lambda i: (i, 0)),
            pl.BlockSpec((3 * (K - 1), npf), lambda i: (0, 0)),
            pl.BlockSpec((1, npf), lambda i: (0, 0)),
        ],
        out_specs=[
            pl.BlockSpec((MB, npf), lambda i: (i, 0)),
            pl.BlockSpec((2, npf), lambda i: (0, 0)),
        ],
        out_shape=[
            jax.ShapeDtypeStruct((N, npf), jnp.float32),
            jax.ShapeDtypeStruct((2, npf), jnp.float32),
        ],
        interpret=interpret,
    )(selpc, W1p, b1)

    out = pl.pallas_call(
        _mlp2_body,
        grid=(nblk,),
        in_specs=[
            pl.BlockSpec((2, npf), lambda i: (0, 0)),
            pl.BlockSpec((MB, npf), lambda i: (i, 0)),
            pl.BlockSpec((1, npf), lambda i: (0, 0)),
            pl.BlockSpec((1, npf), lambda i: (0, 0)),
            pl.BlockSpec((npf, npf), lambda i: (0, 0)),
            pl.BlockSpec((1, npf), lambda i: (0, 0)),
        ],
        out_specs=pl.BlockSpec((MB, npf), lambda i: (i, 0)),
        out_shape=jax.ShapeDtypeStruct((N, npf), jnp.float32),
        interpret=interpret,
    )(sums, h, gamma, beta, W2, b2)
    return out


def kernel(indices, W1, b1, gamma, beta, W2, b2):
    coords = indices[:, 1:].astype(jnp.int32)
    colsT = coords.T
    pc = ((coords[:, 0] * 128 + coords[:, 1]) * 128
          + coords[:, 2]).reshape(1, N)
    npf = W1.shape[1]
    W1p = W1.reshape(K - 1, 3, npf).transpose(1, 0, 2).reshape(3 * (K - 1), npf)
    return _run(coords, colsT, pc, W1p,
                b1.reshape(1, npf), gamma.reshape(1, npf),
                beta.reshape(1, npf), W2, b2.reshape(1, npf))
